# baseline (speedup 1.0000x reference)
import jax
import jax.numpy as jnp
from jax.experimental import pallas as pl
from jax.experimental.pallas import tpu as pltpu

SEP = 2
CH = 32
NBUF = 32


def _bounds_kernel(ids_ref, mask_ref,
                   b_ref, r0_ref, lo_ref, hi_ref, istitle_ref,
                   last_ref, tvalid_ref, lvalid_ref, tcnt_ref, lcnt_ref,
                   ktot_ref):
    ids = ids_ref[...]
    msk = mask_ref[...]
    B, S = ids.shape
    KMAX = b_ref.shape[1]
    eq = (ids == SEP)
    idx = jax.lax.broadcasted_iota(jnp.int32, (B, S), 1)

    def first_pos(cond):
        big = jnp.where(cond, idx, S)
        m = jnp.min(big, axis=1, keepdims=True)
        return jnp.where(m == S, 0, m).astype(jnp.int32)

    pos1 = first_pos(eq)
    pos2 = first_pos(eq & (idx > pos1))
    pos3 = first_pos(eq & (idx > pos2))
    total = jnp.sum(eq.astype(jnp.int32), axis=1, keepdims=True)
    mask_sum = jnp.sum(msk, axis=1, keepdims=True).astype(jnp.int32)
    has2 = total >= 2
    has3 = total >= 3

    title_end = pos1
    lead_start = pos2 + 1
    lead_end = jnp.where(has3, pos3, mask_sum)

    title_cnt = jnp.maximum(title_end - 1, 0)
    lead_cnt = jnp.maximum(lead_end - lead_start, 0)
    t_valid = (has2 & (title_cnt > 0)).astype(jnp.int32)
    l_valid = (has2 & (lead_cnt > 0)).astype(jnp.int32)

    te_eff = jnp.where(has2, title_end, 0)
    n_t = jnp.maximum((te_eff + (CH - 1)) // CH, 1)
    lstart_a = (lead_start // 8) * 8
    n_l = jnp.where(has2 & (lead_cnt > 0),
                    (lead_end - lstart_a + (CH - 1)) // CH, 0)
    n = n_t + n_l

    incl = n
    sh = 1
    while sh < B:
        incl = incl + jnp.concatenate(
            [jnp.zeros((sh, 1), jnp.int32), incl[:B - sh]], axis=0)
        sh *= 2
    cum_incl = incl
    ktot = jnp.sum(n)

    kidx = jax.lax.broadcasted_iota(jnp.int32, (1, KMAX), 1)
    before = (cum_incl <= kidx).astype(jnp.int32)
    bk = jnp.minimum(jnp.sum(before, axis=0, keepdims=True), B - 1)

    iota_b = jax.lax.broadcasted_iota(jnp.int32, (B, 1), 0)
    onehot = (iota_b == bk).astype(jnp.int32)

    def gather(col):
        return jnp.sum(col * onehot, axis=0, keepdims=True)

    tend_k = gather(title_end)
    lstart_k = gather(lead_start)
    lend_k = gather(lead_end)
    nt_k = gather(n_t)
    n_k = gather(n)
    cumex_k = jnp.sum(n * before, axis=0, keepdims=True)
    lstarta_k = gather(lstart_a)

    ik = kidx - cumex_k
    is_title = ik < nt_k
    base = jnp.where(is_title, ik * CH, lstarta_k + (ik - nt_k) * CH)
    lo = jnp.where(is_title, jnp.maximum(base, 1),
                   jnp.maximum(base, lstart_k))
    hi = jnp.minimum(jnp.where(is_title, tend_k, lend_k), base + CH)
    r0 = jnp.clip(base, 0, S - CH)

    b_ref[...] = bk
    r0_ref[...] = r0
    lo_ref[...] = lo
    hi_ref[...] = hi
    istitle_ref[...] = is_title.astype(jnp.int32)
    last_ref[...] = ((ik == n_k - 1) & (kidx < ktot)).astype(jnp.int32)
    tvalid_ref[...] = gather(t_valid)
    lvalid_ref[...] = gather(l_valid)
    tcnt_ref[...] = gather(title_cnt).astype(jnp.float32)
    lcnt_ref[...] = gather(lead_cnt).astype(jnp.float32)
    ktot_ref[...] = (((ktot + 3) // 4) * 4).reshape(1, 1)


def _pool_kernel(b_ref, r0_ref, lo_ref, hi_ref, istitle_ref,
                 last_ref, tvalid_ref, lvalid_ref, tcnt_ref, lcnt_ref,
                 ktot_ref, h_ref, title_ref, lead_ref,
                 buf_ref, acc_ref, fb_ref, sems, fb_sem):
    B = h_ref.shape[0]
    S = h_ref.shape[1]
    H = h_ref.shape[2]
    ktot = ktot_ref[0, 0]

    def start_copy(slot, b, r0):
        pltpu.make_async_copy(
            h_ref.at[b, pl.ds(pl.multiple_of(r0, 8), CH), :],
            buf_ref.at[slot],
            sems.at[slot],
        ).start()

    fb_copy = pltpu.make_async_copy(h_ref.at[:, 0:1, :], fb_ref, fb_sem)
    fb_copy.start()

    for j in range(NBUF):
        start_copy(j, b_ref[0, j], r0_ref[0, j])

    acc_ref[...] = jnp.zeros_like(acc_ref)
    fb_copy.wait()

    def step(k):
        slot = k % NBUF
        pltpu.make_async_copy(
            h_ref.at[0, pl.ds(0, CH), :], buf_ref.at[slot], sems.at[slot]
        ).wait()

        lo = lo_ref[0, k]
        hi = hi_ref[0, k]
        buf = buf_ref[slot]
        ridx = r0_ref[0, k] + jax.lax.broadcasted_iota(
            jnp.int32, (CH, 1), 0)
        w = ((ridx >= lo) & (ridx < hi)).astype(jnp.float32)
        p = buf * w
        parts = [p[8 * j:8 * (j + 1)] for j in range(CH // 8)]
        while len(parts) > 1:
            parts = [a + b for a, b in
                     zip(parts[::2], parts[1::2])]
        p8 = parts[0]

        row8 = pl.multiple_of((1 - istitle_ref[0, k]) * 8, 8)
        acc_ref[pl.ds(row8, 8), :] += p8

        @pl.when(last_ref[0, k] > 0)
        def _():
            cb = b_ref[0, k]
            t_sum = jnp.sum(acc_ref[0:8, :], axis=0, keepdims=True)
            l_sum = jnp.sum(acc_ref[8:16, :], axis=0, keepdims=True)
            t_mean = t_sum / jnp.maximum(tcnt_ref[0, k], 1.0)
            l_mean = l_sum / jnp.maximum(lcnt_ref[0, k], 1.0)
            fb = fb_ref[cb]
            title_ref[pl.ds(cb, 1), :] = jnp.where(
                tvalid_ref[0, k] > 0, t_mean, fb)
            lead_ref[pl.ds(cb, 1), :] = jnp.where(
                lvalid_ref[0, k] > 0, l_mean, fb)
            acc_ref[...] = jnp.zeros_like(acc_ref)

        kn = k + NBUF
        start_copy(slot, b_ref[0, kn], r0_ref[0, kn])

    def body(m, _):
        step(4 * m)
        step(4 * m + 1)
        step(4 * m + 2)
        step(4 * m + 3)
        return 0

    jax.lax.fori_loop(0, ktot // 4, body, 0)

    for j in range(NBUF):
        pltpu.make_async_copy(
            h_ref.at[0, pl.ds(0, CH), :], buf_ref.at[j], sems.at[j]
        ).wait()


@jax.jit
def kernel(hidden_states, input_ids, attention_mask):
    B, S, H = hidden_states.shape
    KMAX = B * 2 * (S // CH) + NBUF + 2
    KMAX = ((KMAX + 127) // 128) * 128
    ids = input_ids.astype(jnp.int32)
    msk = attention_mask.astype(jnp.int32)

    wl_i32 = jax.ShapeDtypeStruct((1, KMAX), jnp.int32)
    wl_f32 = jax.ShapeDtypeStruct((1, KMAX), jnp.float32)
    wl = pl.pallas_call(
        _bounds_kernel,
        out_shape=[wl_i32] * 8 + [wl_f32] * 2 +
                  [jax.ShapeDtypeStruct((1, 1), jnp.int32)],
    )(ids, msk)

    title, lead = pl.pallas_call(
        _pool_kernel,
        in_specs=[pl.BlockSpec(memory_space=pltpu.SMEM)] * 11 +
                 [pl.BlockSpec(memory_space=pl.ANY)],
        out_specs=[
            pl.BlockSpec(memory_space=pltpu.VMEM),
            pl.BlockSpec(memory_space=pltpu.VMEM),
        ],
        out_shape=[
            jax.ShapeDtypeStruct((B, H), jnp.float32),
            jax.ShapeDtypeStruct((B, H), jnp.float32),
        ],
        scratch_shapes=[
            pltpu.VMEM((NBUF, CH, H), jnp.float32),
            pltpu.VMEM((16, H), jnp.float32),
            pltpu.VMEM((B, 1, H), jnp.float32),
            pltpu.SemaphoreType.DMA((NBUF,)),
            pltpu.SemaphoreType.DMA,
        ],
    )(*wl, hidden_states)
    return title, lead


# speedup vs baseline: 1.0586x; 1.0586x over previous
import jax
import jax.numpy as jnp
from jax.experimental import pallas as pl
from jax.experimental.pallas import tpu as pltpu

SEP = 2
CH = 32
NBUF = 32


def _bounds_kernel(ids_ref, mask_ref,
                   b_ref, r0_ref, lo_ref, hi_ref, istitle_ref,
                   last_ref, tvalid_ref, lvalid_ref, tcnt_ref, lcnt_ref,
                   ktot_ref):
    ids = ids_ref[...]
    msk = mask_ref[...]
    B, S = ids.shape
    KMAX = b_ref.shape[1]
    eq = (ids == SEP)
    idx = jax.lax.broadcasted_iota(jnp.int32, (B, S), 1)

    def first_pos(cond):
        big = jnp.where(cond, idx, S)
        m = jnp.min(big, axis=1, keepdims=True)
        return jnp.where(m == S, 0, m).astype(jnp.int32)

    pos1 = first_pos(eq)
    pos2 = first_pos(eq & (idx > pos1))
    pos3 = first_pos(eq & (idx > pos2))
    total = jnp.sum(eq.astype(jnp.int32), axis=1, keepdims=True)
    mask_sum = jnp.sum(msk, axis=1, keepdims=True).astype(jnp.int32)
    has2 = total >= 2
    has3 = total >= 3

    title_end = pos1
    lead_start = pos2 + 1
    lead_end = jnp.where(has3, pos3, mask_sum)

    title_cnt = jnp.maximum(title_end - 1, 0)
    lead_cnt = jnp.maximum(lead_end - lead_start, 0)
    t_valid = (has2 & (title_cnt > 0)).astype(jnp.int32)
    l_valid = (has2 & (lead_cnt > 0)).astype(jnp.int32)

    te_eff = jnp.where(has2, title_end, 0)
    n_t = jnp.maximum((te_eff + (CH - 1)) // CH, 1)
    lstart_a = (lead_start // 8) * 8
    n_l = jnp.where(has2 & (lead_cnt > 0),
                    (lead_end - lstart_a + (CH - 1)) // CH, 0)
    n = n_t + n_l

    incl = n
    sh = 1
    while sh < B:
        incl = incl + jnp.concatenate(
            [jnp.zeros((sh, 1), jnp.int32), incl[:B - sh]], axis=0)
        sh *= 2
    cum_incl = incl
    ktot = jnp.sum(n)

    kidx = jax.lax.broadcasted_iota(jnp.int32, (1, KMAX), 1)
    before = (cum_incl <= kidx).astype(jnp.int32)
    bk = jnp.minimum(jnp.sum(before, axis=0, keepdims=True), B - 1)

    iota_b = jax.lax.broadcasted_iota(jnp.int32, (B, 1), 0)
    onehot = (iota_b == bk).astype(jnp.int32)

    def gather(col):
        return jnp.sum(col * onehot, axis=0, keepdims=True)

    tend_k = gather(title_end)
    lstart_k = gather(lead_start)
    lend_k = gather(lead_end)
    nt_k = gather(n_t)
    n_k = gather(n)
    cumex_k = jnp.sum(n * before, axis=0, keepdims=True)
    lstarta_k = gather(lstart_a)

    ik = kidx - cumex_k
    is_title = ik < nt_k
    base = jnp.where(is_title, ik * CH, lstarta_k + (ik - nt_k) * CH)
    lo = jnp.where(is_title, jnp.maximum(base, 1),
                   jnp.maximum(base, lstart_k))
    hi = jnp.minimum(jnp.where(is_title, tend_k, lend_k), base + CH)
    r0 = jnp.clip(base, 0, S - CH)

    b_ref[...] = bk
    r0_ref[...] = r0
    lo_ref[...] = lo
    hi_ref[...] = hi
    istitle_ref[...] = is_title.astype(jnp.int32)
    last_ref[...] = ((ik == n_k - 1) & (kidx < ktot)).astype(jnp.int32)
    iota_row = jax.lax.broadcasted_iota(jnp.int32, (1, B), 1)
    onehot_b = (iota_b == iota_row).astype(jnp.int32)

    def to_row(col):
        return jnp.sum(col * onehot_b, axis=0, keepdims=True)

    tvalid_ref[...] = to_row(t_valid)
    lvalid_ref[...] = to_row(l_valid)
    tcnt_ref[...] = to_row(title_cnt).astype(jnp.float32)
    lcnt_ref[...] = to_row(lead_cnt).astype(jnp.float32)
    ktot_ref[...] = (((ktot + 1) // 2) * 2).reshape(1, 1)


def _pool_kernel(b_ref, r0_ref, lo_ref, hi_ref, istitle_ref,
                 last_ref, tvalid_ref, lvalid_ref, tcnt_ref, lcnt_ref,
                 ktot_ref, h_ref, title_ref, lead_ref,
                 buf_ref, acc_ref, fb_ref, sems, fb_sem):
    B = h_ref.shape[0]
    S = h_ref.shape[1]
    H = h_ref.shape[2]
    ktot = ktot_ref[0, 0]

    def start_copy(slot, b, r0):
        pltpu.make_async_copy(
            h_ref.at[b, pl.ds(pl.multiple_of(r0, 8), CH), :],
            buf_ref.at[slot],
            sems.at[slot],
        ).start()

    fb_copy = pltpu.make_async_copy(h_ref.at[:, 0:1, :], fb_ref, fb_sem)
    fb_copy.start()

    for j in range(NBUF):
        start_copy(j, b_ref[0, j], r0_ref[0, j])

    acc_ref[...] = jnp.zeros_like(acc_ref)
    fb_copy.wait()

    def step(k):
        slot = k % NBUF
        pltpu.make_async_copy(
            h_ref.at[0, pl.ds(0, CH), :], buf_ref.at[slot], sems.at[slot]
        ).wait()

        lo = lo_ref[0, k]
        hi = hi_ref[0, k]
        buf = buf_ref[slot]
        ridx = r0_ref[0, k] + jax.lax.broadcasted_iota(
            jnp.int32, (CH, 1), 0)
        w = ((ridx >= lo) & (ridx < hi)).astype(jnp.float32)
        p = buf * w
        parts = [p[8 * j:8 * (j + 1)] for j in range(CH // 8)]
        while len(parts) > 1:
            parts = [a + b for a, b in
                     zip(parts[::2], parts[1::2])]
        p8 = parts[0]

        row8 = pl.multiple_of((1 - istitle_ref[0, k]) * 8, 8)
        acc_ref[pl.ds(row8, 8), :] += p8

        @pl.when(last_ref[0, k] > 0)
        def _():
            cb = b_ref[0, k]
            t_sum = jnp.sum(acc_ref[0:8, :], axis=0, keepdims=True)
            l_sum = jnp.sum(acc_ref[8:16, :], axis=0, keepdims=True)
            t_mean = t_sum / jnp.maximum(tcnt_ref[0, cb], 1.0)
            l_mean = l_sum / jnp.maximum(lcnt_ref[0, cb], 1.0)
            fb = fb_ref[cb]
            title_ref[pl.ds(cb, 1), :] = jnp.where(
                tvalid_ref[0, cb] > 0, t_mean, fb)
            lead_ref[pl.ds(cb, 1), :] = jnp.where(
                lvalid_ref[0, cb] > 0, l_mean, fb)
            acc_ref[...] = jnp.zeros_like(acc_ref)

        kn = k + NBUF
        start_copy(slot, b_ref[0, kn], r0_ref[0, kn])

    def body(m, _):
        step(2 * m)
        step(2 * m + 1)
        return 0

    jax.lax.fori_loop(0, ktot // 2, body, 0)

    for j in range(NBUF):
        pltpu.make_async_copy(
            h_ref.at[0, pl.ds(0, CH), :], buf_ref.at[j], sems.at[j]
        ).wait()


@jax.jit
def kernel(hidden_states, input_ids, attention_mask):
    B, S, H = hidden_states.shape
    KMAX = B * 2 * (S // CH) + NBUF + 2
    KMAX = ((KMAX + 127) // 128) * 128
    ids = input_ids.astype(jnp.int32)
    msk = attention_mask.astype(jnp.int32)

    wl_i32 = jax.ShapeDtypeStruct((1, KMAX), jnp.int32)
    ex_i32 = jax.ShapeDtypeStruct((1, B), jnp.int32)
    ex_f32 = jax.ShapeDtypeStruct((1, B), jnp.float32)
    wl = pl.pallas_call(
        _bounds_kernel,
        out_shape=[wl_i32] * 6 + [ex_i32] * 2 + [ex_f32] * 2 +
                  [jax.ShapeDtypeStruct((1, 1), jnp.int32)],
    )(ids, msk)

    title, lead = pl.pallas_call(
        _pool_kernel,
        in_specs=[pl.BlockSpec(memory_space=pltpu.SMEM)] * 11 +
                 [pl.BlockSpec(memory_space=pl.ANY)],
        out_specs=[
            pl.BlockSpec(memory_space=pltpu.VMEM),
            pl.BlockSpec(memory_space=pltpu.VMEM),
        ],
        out_shape=[
            jax.ShapeDtypeStruct((B, H), jnp.float32),
            jax.ShapeDtypeStruct((B, H), jnp.float32),
        ],
        scratch_shapes=[
            pltpu.VMEM((NBUF, CH, H), jnp.float32),
            pltpu.VMEM((16, H), jnp.float32),
            pltpu.VMEM((B, 1, H), jnp.float32),
            pltpu.SemaphoreType.DMA((NBUF,)),
            pltpu.SemaphoreType.DMA,
        ],
    )(*wl, hidden_states)
    return title, lead


# speedup vs baseline: 1.1018x; 1.0408x over previous
import jax
import jax.numpy as jnp
from jax.experimental import pallas as pl
from jax.experimental.pallas import tpu as pltpu

SEP = 2
CH = 32
NBUF = 32


def _bounds_kernel(ids_ref, mask_ref,
                   b_ref, r0_ref, lo_ref, hi_ref, istitle_ref,
                   last_ref, tvalid_ref, lvalid_ref, tcnt_ref, lcnt_ref,
                   ktot_ref):
    ids = ids_ref[...]
    msk = mask_ref[...]
    B, S = ids.shape
    KMAX = b_ref.shape[1]
    eq = (ids == SEP)
    idx = jax.lax.broadcasted_iota(jnp.int32, (B, S), 1)

    def first_pos(cond):
        big = jnp.where(cond, idx, S)
        m = jnp.min(big, axis=1, keepdims=True)
        return jnp.where(m == S, 0, m).astype(jnp.int32)

    pos1 = first_pos(eq)
    pos2 = first_pos(eq & (idx > pos1))
    pos3 = first_pos(eq & (idx > pos2))
    total = jnp.sum(eq.astype(jnp.int32), axis=1, keepdims=True)
    mask_sum = jnp.sum(msk, axis=1, keepdims=True).astype(jnp.int32)
    has2 = total >= 2
    has3 = total >= 3

    title_end = pos1
    lead_start = pos2 + 1
    lead_end = jnp.where(has3, pos3, mask_sum)

    title_cnt = jnp.maximum(title_end - 1, 0)
    lead_cnt = jnp.maximum(lead_end - lead_start, 0)
    t_valid = (has2 & (title_cnt > 0)).astype(jnp.int32)
    l_valid = (has2 & (lead_cnt > 0)).astype(jnp.int32)

    te_eff = jnp.where(has2, title_end, 0)
    n_t = jnp.maximum((te_eff + (CH - 1)) // CH, 1)
    lstart_a = (lead_start // 8) * 8
    n_l = jnp.where(has2 & (lead_cnt > 0),
                    (lead_end - lstart_a + (CH - 1)) // CH, 0)
    n = n_t + n_l

    incl = n
    sh = 1
    while sh < B:
        incl = incl + jnp.concatenate(
            [jnp.zeros((sh, 1), jnp.int32), incl[:B - sh]], axis=0)
        sh *= 2
    cum_incl = incl
    ktot = jnp.sum(n)

    kidx = jax.lax.broadcasted_iota(jnp.int32, (1, KMAX), 1)
    before = (cum_incl <= kidx).astype(jnp.int32)
    bk = jnp.minimum(jnp.sum(before, axis=0, keepdims=True), B - 1)

    iota_b = jax.lax.broadcasted_iota(jnp.int32, (B, 1), 0)
    onehot = (iota_b == bk).astype(jnp.int32)

    def gather(col):
        return jnp.sum(col * onehot, axis=0, keepdims=True)

    tend_k = gather(title_end)
    lstart_k = gather(lead_start)
    lend_k = gather(lead_end)
    nt_k = gather(n_t)
    n_k = gather(n)
    cumex_k = jnp.sum(n * before, axis=0, keepdims=True)
    lstarta_k = gather(lstart_a)

    ik = kidx - cumex_k
    is_title = ik < nt_k
    base = jnp.where(is_title, ik * CH, lstarta_k + (ik - nt_k) * CH)
    lo = jnp.where(is_title, jnp.maximum(base, 1),
                   jnp.maximum(base, lstart_k))
    hi = jnp.minimum(jnp.where(is_title, tend_k, lend_k), base + CH)
    r0 = jnp.clip(base, 0, S - CH)

    b_ref[...] = bk
    r0_ref[...] = r0
    lo_ref[...] = lo
    hi_ref[...] = hi
    istitle_ref[...] = is_title.astype(jnp.int32)
    last_ref[...] = ((ik == n_k - 1) & (kidx < ktot)).astype(jnp.int32)
    iota_row = jax.lax.broadcasted_iota(jnp.int32, (1, B), 1)
    onehot_b = (iota_b == iota_row).astype(jnp.int32)

    def to_row(col):
        return jnp.sum(col * onehot_b, axis=0, keepdims=True)

    tvalid_ref[...] = to_row(t_valid)
    lvalid_ref[...] = to_row(l_valid)
    tcnt_ref[...] = to_row(title_cnt).astype(jnp.float32)
    lcnt_ref[...] = to_row(lead_cnt).astype(jnp.float32)
    ktot_ref[...] = (((ktot + 1) // 2) * 2).reshape(1, 1)


def _pool_kernel(b_ref, r0_ref, lo_ref, hi_ref, istitle_ref,
                 last_ref, tvalid_ref, lvalid_ref, tcnt_ref, lcnt_ref,
                 ktot_ref, h_ref, title_ref, lead_ref,
                 buf_ref, acc_ref, fb_ref, sems, fb_sem):
    B = h_ref.shape[0]
    S = h_ref.shape[1]
    H = h_ref.shape[2]
    ktot = ktot_ref[0, 0]

    def start_copy(slot, b, r0):
        pltpu.make_async_copy(
            h_ref.at[b, pl.ds(pl.multiple_of(r0, 8), CH), :],
            buf_ref.at[slot],
            sems.at[slot],
        ).start()

    fb_copy = pltpu.make_async_copy(h_ref.at[:, 0:1, :], fb_ref, fb_sem)
    fb_copy.start()

    for j in range(NBUF):
        start_copy(j, b_ref[0, j], r0_ref[0, j])

    acc_ref[...] = jnp.zeros_like(acc_ref)
    fb_copy.wait()

    def step(k):
        slot = k % NBUF
        pltpu.make_async_copy(
            h_ref.at[0, pl.ds(0, CH), :], buf_ref.at[slot], sems.at[slot]
        ).wait()

        lo = lo_ref[0, k]
        hi = hi_ref[0, k]
        buf = buf_ref[slot]
        ridx = r0_ref[0, k] + jax.lax.broadcasted_iota(
            jnp.int32, (CH, 1), 0)
        w = ((ridx >= lo) & (ridx < hi)).astype(jnp.float32)
        p = buf * w
        parts = [p[8 * j:8 * (j + 1)] for j in range(CH // 8)]
        while len(parts) > 1:
            parts = [a + b for a, b in
                     zip(parts[::2], parts[1::2])]
        p8 = parts[0]

        row8 = pl.multiple_of((1 - istitle_ref[0, k]) * 8, 8)
        acc_ref[pl.ds(row8, 8), :] += p8

        @pl.when(last_ref[0, k] > 0)
        def _():
            cb = b_ref[0, k]
            t_sum = jnp.sum(acc_ref[0:8, :], axis=0, keepdims=True)
            l_sum = jnp.sum(acc_ref[8:16, :], axis=0, keepdims=True)
            t_mean = t_sum / jnp.maximum(tcnt_ref[0, cb], 1.0)
            l_mean = l_sum / jnp.maximum(lcnt_ref[0, cb], 1.0)
            fb = fb_ref[cb]
            title_ref[pl.ds(cb, 1), :] = jnp.where(
                tvalid_ref[0, cb] > 0, t_mean, fb)
            lead_ref[pl.ds(cb, 1), :] = jnp.where(
                lvalid_ref[0, cb] > 0, l_mean, fb)
            acc_ref[...] = jnp.zeros_like(acc_ref)

        kn = k + NBUF
        start_copy(slot, b_ref[0, kn], r0_ref[0, kn])

    def body(m, _):
        step(2 * m)
        step(2 * m + 1)
        return 0

    jax.lax.fori_loop(0, ktot // 2, body, 0)

    for j in range(NBUF):
        pltpu.make_async_copy(
            h_ref.at[0, pl.ds(0, CH), :], buf_ref.at[j], sems.at[j]
        ).wait()


@jax.jit
def kernel(hidden_states, input_ids, attention_mask):
    B, S, H = hidden_states.shape
    KMAX = B * (S // CH + 3) + NBUF + 2
    KMAX = ((KMAX + 127) // 128) * 128
    ids = input_ids.astype(jnp.int32)
    msk = attention_mask.astype(jnp.int32)

    wl_i32 = jax.ShapeDtypeStruct((1, KMAX), jnp.int32)
    ex_i32 = jax.ShapeDtypeStruct((1, B), jnp.int32)
    ex_f32 = jax.ShapeDtypeStruct((1, B), jnp.float32)
    wl = pl.pallas_call(
        _bounds_kernel,
        out_shape=[wl_i32] * 6 + [ex_i32] * 2 + [ex_f32] * 2 +
                  [jax.ShapeDtypeStruct((1, 1), jnp.int32)],
    )(ids, msk)

    title, lead = pl.pallas_call(
        _pool_kernel,
        in_specs=[pl.BlockSpec(memory_space=pltpu.SMEM)] * 11 +
                 [pl.BlockSpec(memory_space=pl.ANY)],
        out_specs=[
            pl.BlockSpec(memory_space=pltpu.VMEM),
            pl.BlockSpec(memory_space=pltpu.VMEM),
        ],
        out_shape=[
            jax.ShapeDtypeStruct((B, H), jnp.float32),
            jax.ShapeDtypeStruct((B, H), jnp.float32),
        ],
        scratch_shapes=[
            pltpu.VMEM((NBUF, CH, H), jnp.float32),
            pltpu.VMEM((16, H), jnp.float32),
            pltpu.VMEM((B, 1, H), jnp.float32),
            pltpu.SemaphoreType.DMA((NBUF,)),
            pltpu.SemaphoreType.DMA,
        ],
    )(*wl, hidden_states)
    return title, lead
